# baseline (speedup 1.0000x reference)
import jax
import jax.numpy as jnp
from jax.experimental import pallas as pl
from jax.experimental.pallas import tpu as pltpu

_EPS = 1e-5


def _make_block_kernel(H, W, C):
    WP = W + 2
    FP = (H + 2) * WP
    FA = H * WP
    FPX = FP + 4
    OFF = WP + 1
    taps = [(ky, kx) for ky in range(3) for kx in range(3)]

    def body(xpad_ref, w1_ref, w2_ref, s1_ref, b1_ref, s2_ref, b2_ref,
             mask_ref, out_ref, xbf_ref, s_ref, ypad_ref):
        xbf_ref[:, 0:FP] = xpad_ref[...].astype(jnp.bfloat16)
        xbf_ref[:, FP:FPX] = jnp.zeros((C, FPX - FP), jnp.bfloat16)

        for t, (ky, kx) in enumerate(taps):
            o = ky * WP + kx
            s_ref[t * C:(t + 1) * C, :] = xbf_ref[:, o:o + FA]
        acc = jnp.dot(w1_ref[...], s_ref[...],
                      preferred_element_type=jnp.float32)

        y = jnp.maximum(acc * s1_ref[...] + b1_ref[...], 0.0) * mask_ref[...]
        ypad_ref[:, 0:OFF] = jnp.zeros((C, OFF), jnp.bfloat16)
        ypad_ref[:, OFF + FA:FPX] = jnp.zeros((C, FPX - OFF - FA),
                                              jnp.bfloat16)
        ypad_ref[:, OFF:OFF + FA] = y.astype(jnp.bfloat16)

        for t, (ky, kx) in enumerate(taps):
            o = ky * WP + kx
            s_ref[t * C:(t + 1) * C, :] = ypad_ref[:, o:o + FA]
        acc2 = jnp.dot(w2_ref[...], s_ref[...],
                       preferred_element_type=jnp.float32)

        o2 = acc2 * s2_ref[...] + b2_ref[...] + xpad_ref[:, OFF:OFF + FA]
        out_ref[...] = jnp.maximum(o2, 0.0)

    return body


def _fold_bn(conv_bias, gamma, beta, mean, var):
    scale = gamma / jnp.sqrt(var + _EPS)
    shift = beta + scale * (conv_bias - mean)
    return scale, shift


def _basic_block(x_nchw, conv1_w, conv1_b, bn1_gamma, bn1_beta, bn1_mean,
                 bn1_var, conv2_w, conv2_b, bn2_gamma, bn2_beta, bn2_mean,
                 bn2_var, interpret=False):
    N, C, H, W = x_nchw.shape
    WP = W + 2
    FP = (H + 2) * WP
    FA = H * WP
    FPX = FP + 4

    xpad = jnp.pad(x_nchw, ((0, 0), (0, 0), (1, 1), (1, 1))).reshape(N, C, FP)

    s1, b1 = _fold_bn(conv1_b, bn1_gamma, bn1_beta, bn1_mean, bn1_var)
    s2, b2 = _fold_bn(conv2_b, bn2_gamma, bn2_beta, bn2_mean, bn2_var)
    s1 = s1.reshape(C, 1).astype(jnp.float32)
    b1 = b1.reshape(C, 1).astype(jnp.float32)
    s2 = s2.reshape(C, 1).astype(jnp.float32)
    b2 = b2.reshape(C, 1).astype(jnp.float32)

    w1t = jnp.transpose(conv1_w.reshape(9, C, C), (2, 0, 1))
    w1t = w1t.reshape(C, 9 * C).astype(jnp.bfloat16)
    w2t = jnp.transpose(conv2_w.reshape(9, C, C), (2, 0, 1))
    w2t = w2t.reshape(C, 9 * C).astype(jnp.bfloat16)

    mask = (jnp.arange(FA) % WP < W).astype(jnp.float32).reshape(1, FA)

    flops = 2 * N * H * W * 9 * (C * C) * 2
    bytes_accessed = (x_nchw.size + N * C * FA) * 4 + (w1t.size + w2t.size) * 2

    out_flat = pl.pallas_call(
        _make_block_kernel(H, W, C),
        out_shape=jax.ShapeDtypeStruct((N, C, FA), jnp.float32),
        grid=(N,),
        in_specs=[
            pl.BlockSpec((None, C, FP), lambda n: (n, 0, 0)),
            pl.BlockSpec((C, 9 * C), lambda n: (0, 0)),
            pl.BlockSpec((C, 9 * C), lambda n: (0, 0)),
            pl.BlockSpec((C, 1), lambda n: (0, 0)),
            pl.BlockSpec((C, 1), lambda n: (0, 0)),
            pl.BlockSpec((C, 1), lambda n: (0, 0)),
            pl.BlockSpec((C, 1), lambda n: (0, 0)),
            pl.BlockSpec((1, FA), lambda n: (0, 0)),
        ],
        out_specs=pl.BlockSpec((None, C, FA), lambda n: (n, 0, 0)),
        scratch_shapes=[
            pltpu.VMEM((C, FPX), jnp.bfloat16),
            pltpu.VMEM((9 * C, FA), jnp.bfloat16),
            pltpu.VMEM((C, FPX), jnp.bfloat16),
        ],
        compiler_params=pltpu.CompilerParams(
            dimension_semantics=("parallel",)),
        cost_estimate=pl.CostEstimate(
            flops=flops, transcendentals=0, bytes_accessed=bytes_accessed),
        interpret=interpret,
    )(xpad, w1t, w2t, s1, b1, s2, b2, mask)

    return out_flat.reshape(N, C, H, WP)[:, :, :, :W]


def kernel(x_nchw, conv1_w, conv1_b, bn1_gamma, bn1_beta, bn1_mean, bn1_var,
           conv2_w, conv2_b, bn2_gamma, bn2_beta, bn2_mean, bn2_var):
    return _basic_block(x_nchw, conv1_w, conv1_b, bn1_gamma, bn1_beta,
                        bn1_mean, bn1_var, conv2_w, conv2_b, bn2_gamma,
                        bn2_beta, bn2_mean, bn2_var)


# speedup vs baseline: 2.1504x; 2.1504x over previous
import jax
import jax.numpy as jnp
from jax.experimental import pallas as pl
from jax.experimental.pallas import tpu as pltpu

_EPS = 1e-5


def _make_block_kernel(H, W, C):
    L = H * W

    def stack_dx(dst_ref, v_bf, m0, m1):
        z = jnp.zeros((C, 1), jnp.bfloat16)
        dst_ref[0:C, :] = jnp.concatenate([v_bf[:, 1:], z], axis=1) * m1
        dst_ref[C:2 * C, :] = v_bf
        dst_ref[2 * C:3 * C, :] = jnp.concatenate([z, v_bf[:, :L - 1]],
                                                  axis=1) * m0

    def combine_dy(R):
        zc = jnp.zeros((C, W), jnp.float32)
        up = jnp.concatenate([R[2 * C:3 * C, W:], zc], axis=1)
        dn = jnp.concatenate([zc, R[0:C, :L - W]], axis=1)
        return R[C:2 * C, :] + up + dn

    def body(x_ref, wa_ref, wb_ref, s1_ref, b1_ref, s2_ref, b2_ref,
             m0_ref, m1_ref, out_ref, bs_ref):
        m0 = m0_ref[...]
        m1 = m1_ref[...]

        stack_dx(bs_ref, x_ref[...].astype(jnp.bfloat16), m0, m1)
        r1 = jnp.dot(wa_ref[...], bs_ref[...],
                     preferred_element_type=jnp.float32)
        y = jnp.maximum(combine_dy(r1) * s1_ref[...] + b1_ref[...], 0.0)

        stack_dx(bs_ref, y.astype(jnp.bfloat16), m0, m1)
        r2 = jnp.dot(wb_ref[...], bs_ref[...],
                     preferred_element_type=jnp.float32)
        o = combine_dy(r2) * s2_ref[...] + b2_ref[...] + x_ref[...]
        out_ref[...] = jnp.maximum(o, 0.0)

    return body


def _fold_bn(conv_bias, gamma, beta, mean, var):
    scale = gamma / jnp.sqrt(var + _EPS)
    shift = beta + scale * (conv_bias - mean)
    return scale, shift


def _pack_weights(w, C):
    wt = jnp.transpose(w, (0, 1, 3, 2))
    rows = [jnp.concatenate([wt[r, 2], wt[r, 1], wt[r, 0]], axis=1)
            for r in range(3)]
    return jnp.concatenate(rows, axis=0).astype(jnp.bfloat16)


def _basic_block(x_nchw, conv1_w, conv1_b, bn1_gamma, bn1_beta, bn1_mean,
                 bn1_var, conv2_w, conv2_b, bn2_gamma, bn2_beta, bn2_mean,
                 bn2_var, interpret=False):
    N, C, H, W = x_nchw.shape
    L = H * W
    x_flat = x_nchw.reshape(N, C, L)

    s1, b1 = _fold_bn(conv1_b, bn1_gamma, bn1_beta, bn1_mean, bn1_var)
    s2, b2 = _fold_bn(conv2_b, bn2_gamma, bn2_beta, bn2_mean, bn2_var)
    s1 = s1.reshape(C, 1).astype(jnp.float32)
    b1 = b1.reshape(C, 1).astype(jnp.float32)
    s2 = s2.reshape(C, 1).astype(jnp.float32)
    b2 = b2.reshape(C, 1).astype(jnp.float32)

    wa = _pack_weights(conv1_w, C)
    wb = _pack_weights(conv2_w, C)

    lane = jnp.arange(L) % W
    m0 = (lane > 0).astype(jnp.bfloat16).reshape(1, L)
    m1 = (lane < W - 1).astype(jnp.bfloat16).reshape(1, L)

    flops = 2 * N * H * W * 9 * (C * C) * 2
    bytes_accessed = 2 * N * C * L * 4 + (wa.size + wb.size) * 2

    out_flat = pl.pallas_call(
        _make_block_kernel(H, W, C),
        out_shape=jax.ShapeDtypeStruct((N, C, L), jnp.float32),
        grid=(N,),
        in_specs=[
            pl.BlockSpec((None, C, L), lambda n: (n, 0, 0)),
            pl.BlockSpec((3 * C, 3 * C), lambda n: (0, 0)),
            pl.BlockSpec((3 * C, 3 * C), lambda n: (0, 0)),
            pl.BlockSpec((C, 1), lambda n: (0, 0)),
            pl.BlockSpec((C, 1), lambda n: (0, 0)),
            pl.BlockSpec((C, 1), lambda n: (0, 0)),
            pl.BlockSpec((C, 1), lambda n: (0, 0)),
            pl.BlockSpec((1, L), lambda n: (0, 0)),
            pl.BlockSpec((1, L), lambda n: (0, 0)),
        ],
        out_specs=pl.BlockSpec((None, C, L), lambda n: (n, 0, 0)),
        scratch_shapes=[
            pltpu.VMEM((3 * C, L), jnp.bfloat16),
        ],
        compiler_params=pltpu.CompilerParams(
            dimension_semantics=("parallel",)),
        cost_estimate=pl.CostEstimate(
            flops=flops, transcendentals=0, bytes_accessed=bytes_accessed),
        interpret=interpret,
    )(x_flat, wa, wb, s1, b1, s2, b2, m0, m1)

    return out_flat.reshape(N, C, H, W)


def kernel(x_nchw, conv1_w, conv1_b, bn1_gamma, bn1_beta, bn1_mean, bn1_var,
           conv2_w, conv2_b, bn2_gamma, bn2_beta, bn2_mean, bn2_var):
    return _basic_block(x_nchw, conv1_w, conv1_b, bn1_gamma, bn1_beta,
                        bn1_mean, bn1_var, conv2_w, conv2_b, bn2_gamma,
                        bn2_beta, bn2_mean, bn2_var)
